# baseline (speedup 1.0000x reference)
import functools

import jax
import jax.numpy as jnp
from jax import lax
from jax.experimental import pallas as pl
from jax.experimental.pallas import tpu as pltpu
from jax.experimental.pallas import tpu_sc as plsc

_VOCAB = 100000
_D = 64
_S = 200
_B = 4096

_NC = 2
_NS = 16
_L = 16
_NW = _NC * _NS
_BPW = _B // _NW
_G = _BPW // _L
_CH = 40

_ROW_BLK = 16384


_NTC = 59392
_SC_BASE = 59040
_NSC = _VOCAB - _SC_BASE
_CHUNK = 2048
_NCHUNK = _NTC // _CHUNK
_NBUF = 4

_VPW = 1280
_RCH = 640
_NRCH = _VPW // _RCH
_GPC = _RCH // _L


def _tw_body(tbl_hbm, w_ref, b_ref, t_hbm, bufs, osts, isem, osem):
    def in_copy(k):
        slot = k % _NBUF
        return pltpu.make_async_copy(
            tbl_hbm.at[pl.ds(k * _CHUNK, _CHUNK), :], bufs.at[slot],
            isem.at[slot])

    def out_copy(k):
        slot = k % _NBUF
        return pltpu.make_async_copy(
            osts.at[slot], t_hbm.at[pl.ds(k * _CHUNK, _CHUNK), :],
            osem.at[slot])

    for k in range(_NBUF):
        in_copy(k).start()
    for k in range(_NCHUNK):
        slot = k % _NBUF
        in_copy(k).wait()
        if k >= _NBUF:
            out_copy(k - _NBUF).wait()
        osts[slot] = jnp.sum(bufs[slot] * w_ref[...], axis=1,
                             keepdims=True) + b_ref[0]
        out_copy(k).start()
        if k + _NBUF < _NCHUNK:
            in_copy(k + _NBUF).start()
    for k in range(_NCHUNK - _NBUF, _NCHUNK):
        out_copy(k).wait()


def _tc_project(table, W, b):
    t2d = pl.pallas_call(
        _tw_body,
        in_specs=[
            pl.BlockSpec(memory_space=pl.ANY),
            pl.BlockSpec((1, _D), lambda: (0, 0)),
            pl.BlockSpec(memory_space=pltpu.SMEM),
        ],
        out_specs=pl.BlockSpec(memory_space=pl.ANY),
        out_shape=jax.ShapeDtypeStruct((_NTC, 1), jnp.float32),
        scratch_shapes=[
            pltpu.VMEM((_NBUF, _CHUNK, _D), jnp.float32),
            pltpu.VMEM((_NBUF, _CHUNK, 1), jnp.float32),
            pltpu.SemaphoreType.DMA((_NBUF,)),
            pltpu.SemaphoreType.DMA((_NBUF,)),
        ],
    )(table, W.reshape(1, _D), b)
    return t2d.reshape(_NTC)


def _tw_sc_body(tbl_hbm, wb_hbm, t_hbm, buf0, buf1, wb_v, ost_v, isem):
    wid = lax.axis_index("s") * _NC + lax.axis_index("c")
    base = _SC_BASE + wid * _VPW
    bufs = (buf0, buf1)

    pltpu.sync_copy(wb_hbm, wb_v)
    b_vec = wb_v[pl.ds(_D * _L, _L)]

    def in_copy(c, slot):
        return pltpu.async_copy(
            tbl_hbm.at[pl.ds((base + c * _RCH) * _D, _RCH * _D)],
            bufs[slot], isem.at[slot])

    in_copy(0, 0)
    in_copy(1, 1)
    iota64 = lax.iota(jnp.int32, _L) * _D

    for c in range(_NRCH):
        slot = c % 2
        buf = bufs[slot]
        pltpu.make_async_copy(tbl_hbm.at[pl.ds(0, _RCH * _D)], buf,
                              isem.at[slot]).wait()

        def body(g, carry, buf=buf):
            base_vec = iota64 + g * (_L * _D)
            accs = [jnp.zeros((_L,), jnp.float32) for _ in range(4)]
            for d in range(_D):
                v = plsc.load_gather(buf, [base_vec + d])
                accs[d % 4] = accs[d % 4] + v * wb_v[pl.ds(d * _L, _L)]
            tot = (accs[0] + accs[1]) + (accs[2] + accs[3]) + b_vec
            ost_v[pl.ds(g * _L, _L)] = tot
            return carry

        lax.fori_loop(0, _GPC, body, 0)
        if c + 2 < _NRCH:
            in_copy(c + 2, slot)
        pltpu.sync_copy(
            ost_v, t_hbm.at[pl.ds(wid * _VPW + c * _RCH, _RCH)])


def _sc_project(table, W, b):
    wb = jnp.tile(jnp.concatenate([W.reshape(_D), b]).reshape(_D + 1, 1),
                  (1, _L))
    mesh = plsc.VectorSubcoreMesh(core_axis_name="c", subcore_axis_name="s")
    f = functools.partial(
        pl.kernel,
        out_type=jax.ShapeDtypeStruct((_NSC,), jnp.float32),
        mesh=mesh,
        scratch_types=[
            pltpu.VMEM((_RCH * _D,), jnp.float32),
            pltpu.VMEM((_RCH * _D,), jnp.float32),
            pltpu.VMEM(((_D + 1) * _L,), jnp.float32),
            pltpu.VMEM((_RCH,), jnp.float32),
            pltpu.SemaphoreType.DMA((2,)),
        ],
        compiler_params=pltpu.CompilerParams(needs_layout_passes=False),
    )(_tw_sc_body)
    return f(table.reshape(_VOCAB * _D), wb.reshape((_D + 1) * _L))


def _rsqrt(z):
    i = lax.bitcast_convert_type(z, jnp.int32)
    y = lax.bitcast_convert_type(
        jnp.int32(0x5F3759DF) - lax.shift_right_arithmetic(i, 1), jnp.float32)
    for _ in range(3):
        y = y * (1.5 - 0.5 * z * y * y)
    return y


def _sc_body(q1_hbm, q2_hbm, tlo_hbm, thi_hbm, out_hbm, t_v, q1_v, q2_v,
             st_v):
    wid = lax.axis_index("s") * _NC + lax.axis_index("c")
    base = wid * _BPW

    pltpu.sync_copy(tlo_hbm, t_v.at[pl.ds(0, _NTC)])
    pltpu.sync_copy(thi_hbm, t_v.at[pl.ds(_SC_BASE, _NSC)])

    zero = jnp.zeros((_L,), jnp.float32)
    accs = (zero,) * (3 * _G)

    for c in range(_S // _CH):
        pltpu.sync_copy(q1_hbm.at[pl.ds(c * _CH, _CH), pl.ds(base, _BPW)],
                        q1_v)
        pltpu.sync_copy(q2_hbm.at[pl.ds(c * _CH, _CH), pl.ds(base, _BPW)],
                        q2_v)

        def body(s, carry, q1_v=q1_v, q2_v=q2_v, t_v=t_v):
            new = list(carry)
            for g in range(_G):
                i1 = q1_v[s, pl.ds(g * _L, _L)]
                i2 = q2_v[s, pl.ds(g * _L, _L)]
                v1 = plsc.load_gather(t_v, [i1])
                v2 = plsc.load_gather(t_v, [i2])
                new[3 * g] = new[3 * g] + v1 * v2
                new[3 * g + 1] = new[3 * g + 1] + v1 * v1
                new[3 * g + 2] = new[3 * g + 2] + v2 * v2
            return tuple(new)

        accs = lax.fori_loop(0, _CH, body, accs)

    for g in range(_G):
        num = accs[3 * g]
        z = jnp.maximum(accs[3 * g + 1] * accs[3 * g + 2], 1e-28)
        denom = jnp.maximum(z * _rsqrt(z), 1e-8)
        cos = num / denom
        st_v[pl.ds(g * _L, _L)] = 1.0 / (1.0 + jnp.exp(-cos))

    pltpu.sync_copy(st_v, out_hbm.at[pl.ds(base, _BPW)])


def _sc_reduce(q1, q2, t_lo, t_hi):
    mesh = plsc.VectorSubcoreMesh(core_axis_name="c", subcore_axis_name="s")
    f = functools.partial(
        pl.kernel,
        out_type=jax.ShapeDtypeStruct((_B,), jnp.float32),
        mesh=mesh,
        scratch_types=[
            pltpu.VMEM((_VOCAB,), jnp.float32),
            pltpu.VMEM((_CH, _BPW), jnp.int32),
            pltpu.VMEM((_CH, _BPW), jnp.int32),
            pltpu.VMEM((_BPW,), jnp.float32),
        ],
        compiler_params=pltpu.CompilerParams(needs_layout_passes=False),
    )(_sc_body)
    return f(q1, q2, t_lo, t_hi)


def kernel(question1, question2, table, W, b):
    t_hi = _sc_project(table, W, b)
    t_lo = _tc_project(table, W, b)
    return _sc_reduce(question1, question2, t_lo, t_hi)


# speedup vs baseline: 1.5033x; 1.5033x over previous
import functools

import jax
import jax.numpy as jnp
from jax import lax
from jax.experimental import pallas as pl
from jax.experimental.pallas import tpu as pltpu
from jax.experimental.pallas import tpu_sc as plsc

_VOCAB = 100000
_D = 64
_S = 200
_B = 4096

_NC = 2
_NS = 16
_L = 16
_NW = _NC * _NS
_BPW = _B // _NW
_G = _BPW // _L
_CH = 40

_ROW_BLK = 16384


_CHUNK = 2500
_NCHUNK = _VOCAB // _CHUNK
_NBUF = 4


def _tw_body(tbl_hbm, w_ref, b_ref, t_hbm, bufs, osts, isem, osem):
    def in_copy(k):
        slot = k % _NBUF
        return pltpu.make_async_copy(
            tbl_hbm.at[pl.ds(k * _CHUNK, _CHUNK), :], bufs.at[slot],
            isem.at[slot])

    def out_copy(k):
        slot = k % _NBUF
        return pltpu.make_async_copy(
            osts.at[slot], t_hbm.at[pl.ds(k * _CHUNK, _CHUNK), :],
            osem.at[slot])

    for k in range(_NBUF):
        in_copy(k).start()
    for k in range(_NCHUNK):
        slot = k % _NBUF
        in_copy(k).wait()
        if k >= _NBUF:
            out_copy(k - _NBUF).wait()
        osts[slot] = jnp.sum(bufs[slot] * w_ref[...], axis=1,
                             keepdims=True) + b_ref[0]
        out_copy(k).start()
        if k + _NBUF < _NCHUNK:
            in_copy(k + _NBUF).start()
    for k in range(_NCHUNK - _NBUF, _NCHUNK):
        out_copy(k).wait()


def _project_table(table, W, b):
    t2d = pl.pallas_call(
        _tw_body,
        in_specs=[
            pl.BlockSpec(memory_space=pl.ANY),
            pl.BlockSpec((1, _D), lambda: (0, 0)),
            pl.BlockSpec(memory_space=pltpu.SMEM),
        ],
        out_specs=pl.BlockSpec(memory_space=pl.ANY),
        out_shape=jax.ShapeDtypeStruct((_VOCAB, 1), jnp.float32),
        scratch_shapes=[
            pltpu.VMEM((_NBUF, _CHUNK, _D), jnp.float32),
            pltpu.VMEM((_NBUF, _CHUNK, 1), jnp.float32),
            pltpu.SemaphoreType.DMA((_NBUF,)),
            pltpu.SemaphoreType.DMA((_NBUF,)),
        ],
    )(table, W.reshape(1, _D), b)
    return t2d.reshape(_VOCAB)


def _rsqrt(z):
    i = lax.bitcast_convert_type(z, jnp.int32)
    y = lax.bitcast_convert_type(
        jnp.int32(0x5F3759DF) - lax.shift_right_arithmetic(i, 1), jnp.float32)
    for _ in range(3):
        y = y * (1.5 - 0.5 * z * y * y)
    return y


_TSL = 6256
_TSL_LAST = _VOCAB - 15 * _TSL


def _sc_body(q1_hbm, q2_hbm, t_hbm, out_hbm, t_v, q1_v, q2_v, st_v, sh_v):
    sid = lax.axis_index("s")
    wid = sid * _NC + lax.axis_index("c")
    base = wid * _BPW

    off = sid * _TSL

    @pl.when(sid < _NS - 1)
    def _():
        pltpu.sync_copy(t_hbm.at[pl.ds(off, _TSL)],
                        t_v.at[pl.ds(off, _TSL)])
        pltpu.sync_copy(t_v.at[pl.ds(off, _TSL)],
                        sh_v.at[pl.ds(off, _TSL)])

    @pl.when(sid == _NS - 1)
    def _():
        pltpu.sync_copy(t_hbm.at[pl.ds(15 * _TSL, _TSL_LAST)],
                        t_v.at[pl.ds(15 * _TSL, _TSL_LAST)])
        pltpu.sync_copy(t_v.at[pl.ds(15 * _TSL, _TSL_LAST)],
                        sh_v.at[pl.ds(15 * _TSL, _TSL_LAST)])

    plsc.subcore_barrier()
    pltpu.sync_copy(sh_v, t_v)

    zero = jnp.zeros((_L,), jnp.float32)
    accs = (zero,) * (3 * _G)

    for c in range(_S // _CH):
        pltpu.sync_copy(q1_hbm.at[pl.ds(c * _CH, _CH), pl.ds(base, _BPW)],
                        q1_v)
        pltpu.sync_copy(q2_hbm.at[pl.ds(c * _CH, _CH), pl.ds(base, _BPW)],
                        q2_v)

        def body(s, carry, q1_v=q1_v, q2_v=q2_v, t_v=t_v):
            new = list(carry)
            for g in range(_G):
                i1 = q1_v[s, pl.ds(g * _L, _L)]
                i2 = q2_v[s, pl.ds(g * _L, _L)]
                v1 = plsc.load_gather(t_v, [i1])
                v2 = plsc.load_gather(t_v, [i2])
                new[3 * g] = new[3 * g] + v1 * v2
                new[3 * g + 1] = new[3 * g + 1] + v1 * v1
                new[3 * g + 2] = new[3 * g + 2] + v2 * v2
            return tuple(new)

        accs = lax.fori_loop(0, _CH, body, accs)

    for g in range(_G):
        num = accs[3 * g]
        z = jnp.maximum(accs[3 * g + 1] * accs[3 * g + 2], 1e-28)
        denom = jnp.maximum(z * _rsqrt(z), 1e-8)
        cos = num / denom
        st_v[pl.ds(g * _L, _L)] = 1.0 / (1.0 + jnp.exp(-cos))

    pltpu.sync_copy(st_v, out_hbm.at[pl.ds(base, _BPW)])


def _sc_reduce(q1, q2, t):
    mesh = plsc.VectorSubcoreMesh(core_axis_name="c", subcore_axis_name="s")
    f = functools.partial(
        pl.kernel,
        out_type=jax.ShapeDtypeStruct((_B,), jnp.float32),
        mesh=mesh,
        scratch_types=[
            pltpu.VMEM((_VOCAB,), jnp.float32),
            pltpu.VMEM((_CH, _BPW), jnp.int32),
            pltpu.VMEM((_CH, _BPW), jnp.int32),
            pltpu.VMEM((_BPW,), jnp.float32),
            pltpu.VMEM_SHARED((_VOCAB,), jnp.float32),
        ],
        compiler_params=pltpu.CompilerParams(needs_layout_passes=False),
    )(_sc_body)
    return f(q1, q2, t)


def kernel(question1, question2, table, W, b):
    t = _project_table(table, W, b)
    return _sc_reduce(question1, question2, t)


# speedup vs baseline: 1.6060x; 1.0683x over previous
import functools

import jax
import jax.numpy as jnp
from jax import lax
from jax.experimental import pallas as pl
from jax.experimental.pallas import tpu as pltpu
from jax.experimental.pallas import tpu_sc as plsc

_VOCAB = 100000
_D = 64
_S = 200
_B = 4096

_NC = 2
_NS = 16
_L = 16
_NW = _NC * _NS
_BPW = _B // _NW
_G = _BPW // _L
_CH = 40

_ROW_BLK = 16384


_CHUNK = 2500
_NCHUNK = _VOCAB // _CHUNK
_NBUF = 4


def _tw_body(tbl_hbm, w_ref, b_ref, t_hbm, bufs, osts, isem, osem):
    def in_copy(k):
        slot = k % _NBUF
        return pltpu.make_async_copy(
            tbl_hbm.at[pl.ds(k * _CHUNK, _CHUNK), :], bufs.at[slot],
            isem.at[slot])

    def out_copy(k):
        slot = k % _NBUF
        return pltpu.make_async_copy(
            osts.at[slot], t_hbm.at[pl.ds(k * _CHUNK, _CHUNK), :],
            osem.at[slot])

    for k in range(_NBUF):
        in_copy(k).start()
    for k in range(_NCHUNK):
        slot = k % _NBUF
        in_copy(k).wait()
        if k >= _NBUF:
            out_copy(k - _NBUF).wait()
        osts[slot] = jnp.sum(bufs[slot] * w_ref[...], axis=1,
                             keepdims=True) + b_ref[0]
        out_copy(k).start()
        if k + _NBUF < _NCHUNK:
            in_copy(k + _NBUF).start()
    for k in range(_NCHUNK - _NBUF, _NCHUNK):
        out_copy(k).wait()


def _project_table(table, W, b):
    t2d = pl.pallas_call(
        _tw_body,
        in_specs=[
            pl.BlockSpec(memory_space=pl.ANY),
            pl.BlockSpec((1, _D), lambda: (0, 0)),
            pl.BlockSpec(memory_space=pltpu.SMEM),
        ],
        out_specs=pl.BlockSpec(memory_space=pl.ANY),
        out_shape=jax.ShapeDtypeStruct((_VOCAB, 1), jnp.float32),
        scratch_shapes=[
            pltpu.VMEM((_NBUF, _CHUNK, _D), jnp.float32),
            pltpu.VMEM((_NBUF, _CHUNK, 1), jnp.float32),
            pltpu.SemaphoreType.DMA((_NBUF,)),
            pltpu.SemaphoreType.DMA((_NBUF,)),
        ],
    )(table, W.reshape(1, _D), b)
    return t2d.reshape(_VOCAB)


def _rsqrt(z):
    i = lax.bitcast_convert_type(z, jnp.int32)
    y = lax.bitcast_convert_type(
        jnp.int32(0x5F3759DF) - lax.shift_right_arithmetic(i, 1), jnp.float32)
    for _ in range(3):
        y = y * (1.5 - 0.5 * z * y * y)
    return y


_TSL = 6256
_TSL_LAST = _VOCAB - 15 * _TSL


def _sc_body(q1_hbm, q2_hbm, t_hbm, out_hbm, t_v, q1_v, q2_v, st_v, sh_v,
             qsem):
    sid = lax.axis_index("s")
    wid = sid * _NC + lax.axis_index("c")
    base = wid * _BPW

    off = sid * _TSL

    @pl.when(sid < _NS - 1)
    def _():
        pltpu.sync_copy(t_hbm.at[pl.ds(off, _TSL)],
                        t_v.at[pl.ds(off, _TSL)])
        pltpu.sync_copy(t_v.at[pl.ds(off, _TSL)],
                        sh_v.at[pl.ds(off, _TSL)])

    @pl.when(sid == _NS - 1)
    def _():
        pltpu.sync_copy(t_hbm.at[pl.ds(15 * _TSL, _TSL_LAST)],
                        t_v.at[pl.ds(15 * _TSL, _TSL_LAST)])
        pltpu.sync_copy(t_v.at[pl.ds(15 * _TSL, _TSL_LAST)],
                        sh_v.at[pl.ds(15 * _TSL, _TSL_LAST)])

    def q_copy(c):
        slot = c % 2
        return (pltpu.make_async_copy(
                    q1_hbm.at[pl.ds(c * _CH, _CH), pl.ds(base, _BPW)],
                    q1_v.at[slot], qsem.at[0, slot]),
                pltpu.make_async_copy(
                    q2_hbm.at[pl.ds(c * _CH, _CH), pl.ds(base, _BPW)],
                    q2_v.at[slot], qsem.at[1, slot]))

    for d in q_copy(0) + q_copy(1):
        d.start()

    plsc.subcore_barrier()
    pltpu.sync_copy(sh_v, t_v)

    zero = jnp.zeros((_L,), jnp.float32)
    accs = (zero,) * (3 * _G)

    for c in range(_S // _CH):
        slot = c % 2
        for d in q_copy(c):
            d.wait()

        def body(s, carry, slot=slot):
            new = list(carry)
            for g in range(_G):
                i1 = q1_v[slot, s, pl.ds(g * _L, _L)]
                i2 = q2_v[slot, s, pl.ds(g * _L, _L)]
                v1 = plsc.load_gather(t_v, [i1])
                v2 = plsc.load_gather(t_v, [i2])
                new[3 * g] = new[3 * g] + v1 * v2
                new[3 * g + 1] = new[3 * g + 1] + v1 * v1
                new[3 * g + 2] = new[3 * g + 2] + v2 * v2
            return tuple(new)

        accs = lax.fori_loop(0, _CH, body, accs)
        if c + 2 < _S // _CH:
            for d in q_copy(c + 2):
                d.start()

    for g in range(_G):
        num = accs[3 * g]
        z = jnp.maximum(accs[3 * g + 1] * accs[3 * g + 2], 1e-28)
        denom = jnp.maximum(z * _rsqrt(z), 1e-8)
        cos = num / denom
        st_v[pl.ds(g * _L, _L)] = 1.0 / (1.0 + jnp.exp(-cos))

    pltpu.sync_copy(st_v, out_hbm.at[pl.ds(base, _BPW)])


def _sc_reduce(q1, q2, t):
    mesh = plsc.VectorSubcoreMesh(core_axis_name="c", subcore_axis_name="s")
    f = functools.partial(
        pl.kernel,
        out_type=jax.ShapeDtypeStruct((_B,), jnp.float32),
        mesh=mesh,
        scratch_types=[
            pltpu.VMEM((_VOCAB,), jnp.float32),
            pltpu.VMEM((2, _CH, _BPW), jnp.int32),
            pltpu.VMEM((2, _CH, _BPW), jnp.int32),
            pltpu.VMEM((_BPW,), jnp.float32),
            pltpu.VMEM_SHARED((_VOCAB,), jnp.float32),
            pltpu.SemaphoreType.DMA((2, 2)),
        ],
        compiler_params=pltpu.CompilerParams(needs_layout_passes=False),
    )(_sc_body)
    return f(q1, q2, t)


def kernel(question1, question2, table, W, b):
    t = _project_table(table, W, b)
    return _sc_reduce(question1, question2, t)


# speedup vs baseline: 1.6166x; 1.0066x over previous
import functools

import jax
import jax.numpy as jnp
from jax import lax
from jax.experimental import pallas as pl
from jax.experimental.pallas import tpu as pltpu
from jax.experimental.pallas import tpu_sc as plsc

_VOCAB = 100000
_D = 64
_S = 200
_B = 4096

_NC = 2
_NS = 16
_L = 16
_NW = _NC * _NS
_BPW = _B // _NW
_G = _BPW // _L
_CH = 40

_ROW_BLK = 16384


_CHUNK = 1250
_NCHUNK = _VOCAB // _CHUNK
_NBUF = 8


def _tw_body(tbl_hbm, w_ref, b_ref, t_hbm, bufs, osts, isem, osem):
    def in_copy(k):
        slot = k % _NBUF
        return pltpu.make_async_copy(
            tbl_hbm.at[pl.ds(k * _CHUNK, _CHUNK), :], bufs.at[slot],
            isem.at[slot])

    def out_copy(k):
        slot = k % _NBUF
        return pltpu.make_async_copy(
            osts.at[slot], t_hbm.at[pl.ds(k * _CHUNK, _CHUNK), :],
            osem.at[slot])

    for k in range(_NBUF):
        in_copy(k).start()
    for k in range(_NCHUNK):
        slot = k % _NBUF
        in_copy(k).wait()
        if k >= _NBUF:
            out_copy(k - _NBUF).wait()
        osts[slot] = jnp.sum(bufs[slot] * w_ref[...], axis=1,
                             keepdims=True) + b_ref[0]
        out_copy(k).start()
        if k + _NBUF < _NCHUNK:
            in_copy(k + _NBUF).start()
    for k in range(_NCHUNK - _NBUF, _NCHUNK):
        out_copy(k).wait()


def _project_table(table, W, b):
    t2d = pl.pallas_call(
        _tw_body,
        in_specs=[
            pl.BlockSpec(memory_space=pl.ANY),
            pl.BlockSpec((1, _D), lambda: (0, 0)),
            pl.BlockSpec(memory_space=pltpu.SMEM),
        ],
        out_specs=pl.BlockSpec(memory_space=pl.ANY),
        out_shape=jax.ShapeDtypeStruct((_VOCAB, 1), jnp.float32),
        scratch_shapes=[
            pltpu.VMEM((_NBUF, _CHUNK, _D), jnp.float32),
            pltpu.VMEM((_NBUF, _CHUNK, 1), jnp.float32),
            pltpu.SemaphoreType.DMA((_NBUF,)),
            pltpu.SemaphoreType.DMA((_NBUF,)),
        ],
    )(table, W.reshape(1, _D), b)
    return t2d.reshape(_VOCAB)


def _rsqrt(z):
    i = lax.bitcast_convert_type(z, jnp.int32)
    y = lax.bitcast_convert_type(
        jnp.int32(0x5F3759DF) - lax.shift_right_arithmetic(i, 1), jnp.float32)
    for _ in range(3):
        y = y * (1.5 - 0.5 * z * y * y)
    return y


_TSL = 6256
_TSL_LAST = _VOCAB - 15 * _TSL


def _sc_body(q1_hbm, q2_hbm, t_hbm, out_hbm, t_v, q1_v, q2_v, st_v, sh_v,
             qsem):
    sid = lax.axis_index("s")
    wid = sid * _NC + lax.axis_index("c")
    base = wid * _BPW

    off = sid * _TSL

    @pl.when(sid < _NS - 1)
    def _():
        pltpu.sync_copy(t_hbm.at[pl.ds(off, _TSL)],
                        t_v.at[pl.ds(off, _TSL)])
        pltpu.sync_copy(t_v.at[pl.ds(off, _TSL)],
                        sh_v.at[pl.ds(off, _TSL)])

    @pl.when(sid == _NS - 1)
    def _():
        pltpu.sync_copy(t_hbm.at[pl.ds(15 * _TSL, _TSL_LAST)],
                        t_v.at[pl.ds(15 * _TSL, _TSL_LAST)])
        pltpu.sync_copy(t_v.at[pl.ds(15 * _TSL, _TSL_LAST)],
                        sh_v.at[pl.ds(15 * _TSL, _TSL_LAST)])

    def q_copy(c):
        slot = c % 2
        return (pltpu.make_async_copy(
                    q1_hbm.at[pl.ds(c * _CH, _CH), pl.ds(base, _BPW)],
                    q1_v.at[slot], qsem.at[0, slot]),
                pltpu.make_async_copy(
                    q2_hbm.at[pl.ds(c * _CH, _CH), pl.ds(base, _BPW)],
                    q2_v.at[slot], qsem.at[1, slot]))

    for d in q_copy(0) + q_copy(1):
        d.start()

    plsc.subcore_barrier()
    pltpu.sync_copy(sh_v, t_v)

    zero = jnp.zeros((_L,), jnp.float32)
    accs = (zero,) * (3 * _G)

    for c in range(_S // _CH):
        slot = c % 2
        for d in q_copy(c):
            d.wait()

        def body(s, carry, slot=slot):
            new = list(carry)
            for g in range(_G):
                i1 = q1_v[slot, s, pl.ds(g * _L, _L)]
                i2 = q2_v[slot, s, pl.ds(g * _L, _L)]
                v1 = plsc.load_gather(t_v, [i1])
                v2 = plsc.load_gather(t_v, [i2])
                new[3 * g] = new[3 * g] + v1 * v2
                new[3 * g + 1] = new[3 * g + 1] + v1 * v1
                new[3 * g + 2] = new[3 * g + 2] + v2 * v2
            return tuple(new)

        accs = lax.fori_loop(0, _CH, body, accs)
        if c + 2 < _S // _CH:
            for d in q_copy(c + 2):
                d.start()

    for g in range(_G):
        num = accs[3 * g]
        z = jnp.maximum(accs[3 * g + 1] * accs[3 * g + 2], 1e-28)
        denom = jnp.maximum(z * _rsqrt(z), 1e-8)
        cos = num / denom
        st_v[pl.ds(g * _L, _L)] = 1.0 / (1.0 + jnp.exp(-cos))

    pltpu.sync_copy(st_v, out_hbm.at[pl.ds(base, _BPW)])


def _sc_reduce(q1, q2, t):
    mesh = plsc.VectorSubcoreMesh(core_axis_name="c", subcore_axis_name="s")
    f = functools.partial(
        pl.kernel,
        out_type=jax.ShapeDtypeStruct((_B,), jnp.float32),
        mesh=mesh,
        scratch_types=[
            pltpu.VMEM((_VOCAB,), jnp.float32),
            pltpu.VMEM((2, _CH, _BPW), jnp.int32),
            pltpu.VMEM((2, _CH, _BPW), jnp.int32),
            pltpu.VMEM((_BPW,), jnp.float32),
            pltpu.VMEM_SHARED((_VOCAB,), jnp.float32),
            pltpu.SemaphoreType.DMA((2, 2)),
        ],
        compiler_params=pltpu.CompilerParams(needs_layout_passes=False),
    )(_sc_body)
    return f(q1, q2, t)


def kernel(question1, question2, table, W, b):
    t = _project_table(table, W, b)
    return _sc_reduce(question1, question2, t)
